# baseline (speedup 1.0000x reference)
import functools

import jax
import jax.numpy as jnp
from jax import lax
from jax.experimental import pallas as pl
from jax.experimental.pallas import tpu as pltpu
from jax.experimental.pallas import tpu_sc as plsc

B, S, D = 1024, 200, 64
N = B * S
NC, NS = 2, 16
NW = NC * NS
NBB = 8
NSQ = 4
BB = B // NBB
SQ = S // NSQ
VP = 500000
L = 16


def _emb_body(idxh_hbm, par_hbm, pos4_hbm, words2_hbm, out3_hbm,
              idxh_v, par_v, pos_v, g0, g1, p_v, sem0, sem1):
    cid = lax.axis_index("c")
    sid = lax.axis_index("s")
    wid = sid * NC + cid
    bb = wid % NBB
    sq = wid // NBB
    pltpu.sync_copy(idxh_hbm.at[wid], idxh_v)
    pltpu.sync_copy(par_hbm.at[wid], par_v)
    pltpu.sync_copy(pos4_hbm.at[sq], pos_v)

    def start_gather(c, buf, sem):
        pltpu.async_copy(words2_hbm.at[idxh_v.at[c]], buf, sem)

    def wait_gather(buf, sem):
        pltpu.make_async_copy(words2_hbm.at[idxh_v.at[0]], buf, sem).wait()

    iota = lax.iota(jnp.int32, L)

    def consume(c, buf, sem):
        wait_gather(buf, sem)
        rowv = [iota + g * L for g in range(BB // L)]
        colb = [par_v[c, pl.ds(g * L, L)] for g in range(BB // L)]

        def feat(cf, carry):
            cfv = jnp.full((L,), cf, jnp.int32)
            posb = plsc.load_gather(
                pos_v, [jnp.full((L,), c, jnp.int32), cfv])
            for g in range(BB // L):
                val = plsc.load_gather(buf, [rowv[g], colb[g] + cf])
                p_v[cf, pl.ds(g * L, L)] = val + posb
            return carry

        lax.fori_loop(0, D, feat, None)

        @pl.when(c + 2 < SQ)
        def _():
            start_gather(c + 2, buf, sem)

        pltpu.sync_copy(p_v, out3_hbm.at[sq * SQ + c, :, pl.ds(bb * BB, BB)])

    start_gather(0, g0, sem0)
    start_gather(1, g1, sem1)

    def step(t, carry):
        consume(2 * t, g0, sem0)
        consume(2 * t + 1, g1, sem1)
        return carry

    lax.fori_loop(0, SQ // 2, step, None)


def kernel(input_ids, words, pos_table):
    idx_r = (input_ids.astype(jnp.int32).T
             .reshape(NSQ, SQ, NBB, BB).transpose(0, 2, 1, 3)
             .reshape(NW, SQ, BB))
    idx_half = idx_r >> 1
    idx_par = (idx_r & 1) << 6
    words2 = words.reshape(VP, 2 * D)
    pos4 = pos_table[:S].reshape(NSQ, SQ, D)
    mesh = plsc.VectorSubcoreMesh(core_axis_name="c", subcore_axis_name="s")
    out3 = pl.kernel(
        _emb_body,
        out_type=jax.ShapeDtypeStruct((S, D, B), jnp.float32),
        mesh=mesh,
        scratch_types=[
            pltpu.VMEM((SQ, BB), jnp.int32),
            pltpu.VMEM((SQ, BB), jnp.int32),
            pltpu.VMEM((SQ, D), jnp.float32),
            pltpu.VMEM((BB, 2 * D), jnp.float32),
            pltpu.VMEM((BB, 2 * D), jnp.float32),
            pltpu.VMEM((D, BB), jnp.float32),
            pltpu.SemaphoreType.DMA,
            pltpu.SemaphoreType.DMA,
        ],
        compiler_params=pltpu.CompilerParams(use_tc_tiling_on_sc=True,
                                             needs_layout_passes=False),
    )(idx_half, idx_par, pos4, words2)
    return jnp.transpose(out3, (2, 0, 1))


# speedup vs baseline: 1.0158x; 1.0158x over previous
import functools

import jax
import jax.numpy as jnp
from jax import lax
from jax.experimental import pallas as pl
from jax.experimental.pallas import tpu as pltpu
from jax.experimental.pallas import tpu_sc as plsc

B, S, D = 1024, 200, 64
N = B * S
NC, NS = 2, 16
NW = NC * NS
NBB = 8
NSQ = 4
BB = B // NBB
SQ = S // NSQ
VP = 500000
L = 16


def _emb_body(idxh_hbm, par_hbm, pos4_hbm, words2_hbm, out3_hbm,
              idxh_v, par_v, pos_v, g0, g1, p0, p1, sem0, sem1, so0, so1):
    cid = lax.axis_index("c")
    sid = lax.axis_index("s")
    wid = sid * NC + cid
    bb = wid % NBB
    sq = wid // NBB
    pltpu.sync_copy(idxh_hbm.at[wid], idxh_v)
    pltpu.sync_copy(par_hbm.at[wid], par_v)
    pltpu.sync_copy(pos4_hbm.at[sq], pos_v)

    def start_gather(c, buf, sem):
        pltpu.async_copy(words2_hbm.at[idxh_v.at[c]], buf, sem)

    def wait_gather(buf, sem):
        pltpu.make_async_copy(words2_hbm.at[idxh_v.at[0]], buf, sem).wait()

    def out_slice(c):
        return out3_hbm.at[sq * SQ + c, :, pl.ds(bb * BB, BB)]

    iota = lax.iota(jnp.int32, L)
    rowv = [iota + g * L for g in range(BB // L)]

    def consume(c, buf, sem, p_v, so):
        wait_gather(buf, sem)
        colb = [par_v[c, pl.ds(g * L, L)] for g in range(BB // L)]

        @pl.when(c >= 2)
        def _():
            pltpu.make_async_copy(p_v, out_slice(c), so).wait()

        def feat(cf, carry):
            cfv = jnp.full((L,), cf, jnp.int32)
            posb = plsc.load_gather(
                pos_v, [jnp.full((L,), c, jnp.int32), cfv])
            for g in range(BB // L):
                val = plsc.load_gather(buf, [rowv[g], colb[g] + cf])
                p_v[cf, pl.ds(g * L, L)] = val + posb
            return carry

        lax.fori_loop(0, D, feat, None, unroll=4)

        @pl.when(c + 2 < SQ)
        def _():
            start_gather(c + 2, buf, sem)

        pltpu.async_copy(p_v, out_slice(c), so)

    start_gather(0, g0, sem0)
    start_gather(1, g1, sem1)

    def step(t, carry):
        consume(2 * t, g0, sem0, p0, so0)
        consume(2 * t + 1, g1, sem1, p1, so1)
        return carry

    lax.fori_loop(0, SQ // 2, step, None)
    pltpu.make_async_copy(p0, out_slice(SQ - 2), so0).wait()
    pltpu.make_async_copy(p1, out_slice(SQ - 1), so1).wait()


def kernel(input_ids, words, pos_table):
    idx_r = (input_ids.astype(jnp.int32).T
             .reshape(NSQ, SQ, NBB, BB).transpose(0, 2, 1, 3)
             .reshape(NW, SQ, BB))
    idx_half = idx_r >> 1
    idx_par = (idx_r & 1) << 6
    words2 = words.reshape(VP, 2 * D)
    pos4 = pos_table[:S].reshape(NSQ, SQ, D)
    mesh = plsc.VectorSubcoreMesh(core_axis_name="c", subcore_axis_name="s")
    out3 = pl.kernel(
        _emb_body,
        out_type=jax.ShapeDtypeStruct((S, D, B), jnp.float32),
        mesh=mesh,
        scratch_types=[
            pltpu.VMEM((SQ, BB), jnp.int32),
            pltpu.VMEM((SQ, BB), jnp.int32),
            pltpu.VMEM((SQ, D), jnp.float32),
            pltpu.VMEM((BB, 2 * D), jnp.float32),
            pltpu.VMEM((BB, 2 * D), jnp.float32),
            pltpu.VMEM((D, BB), jnp.float32),
            pltpu.VMEM((D, BB), jnp.float32),
            pltpu.SemaphoreType.DMA,
            pltpu.SemaphoreType.DMA,
            pltpu.SemaphoreType.DMA,
            pltpu.SemaphoreType.DMA,
        ],
        compiler_params=pltpu.CompilerParams(use_tc_tiling_on_sc=True,
                                             needs_layout_passes=False),
    )(idx_half, idx_par, pos4, words2)
    return jnp.transpose(out3, (2, 0, 1))


# speedup vs baseline: 1.1085x; 1.0913x over previous
import functools

import jax
import jax.numpy as jnp
from jax import lax
from jax.experimental import pallas as pl
from jax.experimental.pallas import tpu as pltpu
from jax.experimental.pallas import tpu_sc as plsc

B, S, D = 1024, 200, 64
N = B * S
NC, NS = 2, 16
NW = NC * NS
PER_W = N // NW
CH = 128
NCH = PER_W // CH
POS2 = 2 * S
NG = 4
NO = 2
UNROLL = 8


def _emb_body(idx_hbm, pos2_hbm, words_hbm, out_hbm,
              idx_v, pos_v, g0, g1, g2, g3, o0, o1,
              sg0, sg1, sg2, sg3, so0, so1):
    gbufs = (g0, g1, g2, g3)
    gsems = (sg0, sg1, sg2, sg3)
    obufs = (o0, o1)
    osems = (so0, so1)
    cid = lax.axis_index("c")
    sid = lax.axis_index("s")
    wid = sid * NC + cid
    base = wid * PER_W
    pltpu.sync_copy(idx_hbm.at[pl.ds(base, PER_W)], idx_v)
    pltpu.sync_copy(pos2_hbm, pos_v)

    def start_gather(c, k):
        pltpu.async_copy(words_hbm.at[idx_v.at[pl.ds(c * CH, CH)]],
                         gbufs[k], gsems[k])

    def wait_gather(k):
        pltpu.make_async_copy(words_hbm.at[idx_v.at[pl.ds(0, CH)]],
                              gbufs[k], gsems[k]).wait()

    def out_slice(c):
        return out_hbm.at[pl.ds(base + c * CH, CH)]

    def consume(c, k, j):
        wait_gather(k)
        o_v = obufs[j]

        @pl.when(c >= NO)
        def _():
            pltpu.make_async_copy(o_v, out_slice(c), osems[j]).wait()

        rbase = lax.rem(c * CH, S)

        def add_row(r, carry):
            prow = rbase + r
            for jj in range(4):
                sl = pl.ds(jj * 16, 16)
                o_v[r, sl] = gbufs[k][r, sl] + pos_v[prow, sl]
            return carry

        lax.fori_loop(0, CH, add_row, None, unroll=UNROLL)

        @pl.when(c + NG < NCH)
        def _():
            start_gather(c + NG, k)

        pltpu.async_copy(o_v, out_slice(c), osems[j])

    for k in range(NG):
        start_gather(k, k)

    def step(t, carry):
        c0 = NG * t
        for k in range(NG):
            consume(c0 + k, k, k % NO)
        return carry

    lax.fori_loop(0, NCH // NG, step, None)
    consume(jnp.int32(NCH - 2), 0, 0)
    consume(jnp.int32(NCH - 1), 1, 1)
    pltpu.make_async_copy(obufs[0], out_slice(NCH - 2), osems[0]).wait()
    pltpu.make_async_copy(obufs[1], out_slice(NCH - 1), osems[1]).wait()


def kernel(input_ids, words, pos_table):
    idx = input_ids.reshape(-1).astype(jnp.int32)
    pos2 = jnp.concatenate([pos_table[:S], pos_table[:S]], axis=0)
    mesh = plsc.VectorSubcoreMesh(core_axis_name="c", subcore_axis_name="s")
    out = pl.kernel(
        _emb_body,
        out_type=jax.ShapeDtypeStruct((N, D), jnp.float32),
        mesh=mesh,
        scratch_types=(
            [pltpu.VMEM((PER_W,), jnp.int32),
             pltpu.VMEM((POS2, D), jnp.float32)]
            + [pltpu.VMEM((CH, D), jnp.float32) for _ in range(NG + NO)]
            + [pltpu.SemaphoreType.DMA for _ in range(NG + NO)]
        ),
        compiler_params=pltpu.CompilerParams(use_tc_tiling_on_sc=False),
    )(idx, pos2, words)
    return out.reshape(B, S, D)
